# baseline (speedup 1.0000x reference)
import functools

import jax
import jax.numpy as jnp
from jax import lax
from jax.experimental import pallas as pl
from jax.experimental.pallas import tpu as pltpu
from jax.experimental.pallas import tpu_sc as plsc

_B = 4096
_L = 50
_V = 100000
_D = 128
_R = _B * _L

_NW = 32
_BW = _B // _NW
_NCHUNK = _L
_NBUF = 5
_UNROLL = 1


@functools.partial(
    pl.kernel,
    out_type=jax.ShapeDtypeStruct((_R, _D), jnp.float32),
    mesh=plsc.VectorSubcoreMesh(core_axis_name="c", subcore_axis_name="s"),
    scratch_types=[
        pltpu.VMEM((_NCHUNK, _BW), jnp.int32),
        pltpu.VMEM((_BW, _D), jnp.float32),
        pltpu.VMEM((_NBUF, _BW, _D), jnp.float32),
        pltpu.SemaphoreType.DMA((_NBUF,)),
        pltpu.SemaphoreType.DMA((_NBUF,)),
    ],
)
def _sc_gather_add(tok_hbm, emo_hbm, proj_hbm, out_hbm,
                   idx_v, emo_v, rows_v, gsem, ssem):
    w = lax.axis_index("s") * 2 + lax.axis_index("c")
    pltpu.sync_copy(tok_hbm.at[w], idx_v)
    pltpu.sync_copy(emo_hbm.at[pl.ds(w * _BW, _BW)], emo_v)

    def start_gather(j, s):
        pltpu.async_copy(proj_hbm.at[idx_v.at[j]], rows_v.at[s], gsem.at[s])

    for s in range(_NBUF - 1):
        start_gather(s, s)

    def ring_body(jj, _):
        for s in range(_NBUF):
            j = jj * _NBUF + s
            sn = (s + _NBUF - 1) % _NBUF

            @pl.when(j >= 1)
            def _wait_prev_store():
                pltpu.make_async_copy(
                    rows_v.at[sn], out_hbm.at[pl.ds(0, _BW)],
                    ssem.at[sn]).wait()

            @pl.when(j + _NBUF - 1 < _NCHUNK)
            def _next_gather():
                start_gather(j + _NBUF - 1, sn)

            pltpu.make_async_copy(
                proj_hbm.at[idx_v.at[j]], rows_v.at[s], gsem.at[s]).wait()

            def add_body(i, _, s=s):
                for g in range(_UNROLL):
                    r = i * _UNROLL + g
                    for k in range(_D // 16):
                        e = emo_v[r, pl.ds(k * 16, 16)]
                        plsc.addupdate(
                            rows_v.at[s, r, pl.ds(k * 16, 16)], e)
                return _

            lax.fori_loop(0, _BW // _UNROLL, add_body, None)
            pltpu.async_copy(
                rows_v.at[s],
                out_hbm.at[pl.ds(j * _B + w * _BW, _BW)],
                ssem.at[s])
        return _

    lax.fori_loop(0, _NCHUNK // _NBUF, ring_body, None)
    pltpu.make_async_copy(
        rows_v.at[_NBUF - 1], out_hbm.at[pl.ds(0, _BW)],
        ssem.at[_NBUF - 1]).wait()


def _tc_project_table(x, w):
    m = x.shape[0]
    blk = 2000

    def body(x_ref, w_ref, o_ref):
        o_ref[...] = lax.dot_general(
            x_ref[...], w_ref[...], (((1,), (1,)), ((), ())),
            preferred_element_type=jnp.float32)

    return pl.pallas_call(
        body,
        grid=(m // blk,),
        in_specs=[
            pl.BlockSpec((blk, _D), lambda i: (i, 0)),
            pl.BlockSpec((_D, _D), lambda i: (0, 0)),
        ],
        out_specs=pl.BlockSpec((blk, _D), lambda i: (i, 0)),
        out_shape=jax.ShapeDtypeStruct((m, _D), jnp.float32),
    )(x, w)


def _tc_project_emotions(x, w, bias):
    m = x.shape[0]

    def body(x_ref, w_ref, b_ref, o_ref):
        o_ref[...] = lax.dot_general(
            x_ref[...], w_ref[...], (((1,), (1,)), ((), ())),
            preferred_element_type=jnp.float32) + b_ref[...]

    return pl.pallas_call(
        body,
        grid=(1,),
        in_specs=[
            pl.BlockSpec((m, _D), lambda i: (0, 0)),
            pl.BlockSpec((_D, _D), lambda i: (0, 0)),
            pl.BlockSpec((1, _D), lambda i: (0, 0)),
        ],
        out_specs=pl.BlockSpec((m, _D), lambda i: (0, 0)),
        out_shape=jax.ShapeDtypeStruct((m, _D), jnp.float32),
    )(x, w, bias.reshape(1, _D))


def kernel(tokens, emotions, table, W, b):
    tokens = tokens.astype(jnp.int32)
    we = W[:, :_D]
    wm = W[:, _D:]

    proj = _tc_project_table(table, we)
    emo_proj = _tc_project_emotions(emotions, wm, b)
    tok_w = tokens.T.reshape(_L, _NW, _BW).transpose(1, 0, 2)
    out = _sc_gather_add(tok_w, emo_proj, proj)
    return out.reshape(_L, _B, _D).transpose(1, 0, 2)


# speedup vs baseline: 1.1095x; 1.1095x over previous
import functools

import jax
import jax.numpy as jnp
from jax import lax
from jax.experimental import pallas as pl
from jax.experimental.pallas import tpu as pltpu
from jax.experimental.pallas import tpu_sc as plsc

_B = 4096
_L = 50
_V = 100000
_D = 128
_R = _B * _L

_NW = 32
_BW = _B // _NW
_NCHUNK = _L
_NBUF = 5
_UNROLL = 1


@functools.partial(
    pl.kernel,
    out_type=jax.ShapeDtypeStruct((_R, _D), jnp.float32),
    mesh=plsc.VectorSubcoreMesh(core_axis_name="c", subcore_axis_name="s"),
    scratch_types=[
        pltpu.VMEM((_NCHUNK, _BW), jnp.int32),
        pltpu.VMEM((_BW, _D), jnp.float32),
        pltpu.VMEM((_NBUF, _BW, _D), jnp.float32),
        pltpu.SemaphoreType.DMA((_NBUF,)),
        pltpu.SemaphoreType.DMA((_NBUF,)),
    ],
)
def _sc_gather_add(tok_hbm, emo_hbm, proj_hbm, out_hbm,
                   idx_v, emo_v, rows_v, gsem, ssem):
    w = lax.axis_index("s") * 2 + lax.axis_index("c")
    pltpu.sync_copy(tok_hbm.at[w], idx_v)
    pltpu.sync_copy(emo_hbm.at[pl.ds(w * _BW, _BW)], emo_v)

    def start_gather(j, s):
        pltpu.async_copy(proj_hbm.at[idx_v.at[j]], rows_v.at[s], gsem.at[s])

    for s in range(_NBUF - 1):
        start_gather(s, s)

    def ring_body(jj, _):
        for s in range(_NBUF):
            j = jj * _NBUF + s
            sn = (s + _NBUF - 1) % _NBUF

            @pl.when(j >= 1)
            def _wait_prev_store():
                pltpu.make_async_copy(
                    rows_v.at[sn], out_hbm.at[pl.ds(0, _BW)],
                    ssem.at[sn]).wait()

            @pl.when(j + _NBUF - 1 < _NCHUNK)
            def _next_gather():
                start_gather(j + _NBUF - 1, sn)

            pltpu.make_async_copy(
                proj_hbm.at[idx_v.at[j]], rows_v.at[s], gsem.at[s]).wait()

            def add_body(i, _, s=s):
                for g in range(_UNROLL):
                    r = i * _UNROLL + g
                    for k in range(_D // 16):
                        e = emo_v[r, pl.ds(k * 16, 16)]
                        plsc.addupdate(
                            rows_v.at[s, r, pl.ds(k * 16, 16)], e)
                return _

            lax.fori_loop(0, _BW // _UNROLL, add_body, None)
            pltpu.async_copy(
                rows_v.at[s],
                out_hbm.at[pl.ds(j * _B + w * _BW, _BW)],
                ssem.at[s])
        return _

    lax.fori_loop(0, _NCHUNK // _NBUF, ring_body, None)
    pltpu.make_async_copy(
        rows_v.at[_NBUF - 1], out_hbm.at[pl.ds(0, _BW)],
        ssem.at[_NBUF - 1]).wait()


def _tc_project_table(x, w):
    m = x.shape[0]
    blk = 4000

    def body(x_ref, w_ref, o_ref):
        o_ref[...] = lax.dot_general(
            x_ref[...], w_ref[...], (((1,), (1,)), ((), ())),
            preferred_element_type=jnp.float32)

    return pl.pallas_call(
        body,
        grid=(m // blk,),
        in_specs=[
            pl.BlockSpec((blk, _D), lambda i: (i, 0)),
            pl.BlockSpec((_D, _D), lambda i: (0, 0)),
        ],
        out_specs=pl.BlockSpec((blk, _D), lambda i: (i, 0)),
        out_shape=jax.ShapeDtypeStruct((m, _D), jnp.float32),
    )(x, w)


def _tc_project_emotions(x, w, bias):
    m = x.shape[0]

    def body(x_ref, w_ref, b_ref, o_ref):
        o_ref[...] = lax.dot_general(
            x_ref[...], w_ref[...], (((1,), (1,)), ((), ())),
            preferred_element_type=jnp.float32) + b_ref[...]

    return pl.pallas_call(
        body,
        grid=(1,),
        in_specs=[
            pl.BlockSpec((m, _D), lambda i: (0, 0)),
            pl.BlockSpec((_D, _D), lambda i: (0, 0)),
            pl.BlockSpec((1, _D), lambda i: (0, 0)),
        ],
        out_specs=pl.BlockSpec((m, _D), lambda i: (0, 0)),
        out_shape=jax.ShapeDtypeStruct((m, _D), jnp.float32),
    )(x, w, bias.reshape(1, _D))


def kernel(tokens, emotions, table, W, b):
    tokens = tokens.astype(jnp.int32)
    we = W[:, :_D]
    wm = W[:, _D:]

    proj = _tc_project_table(table, we)
    emo_proj = _tc_project_emotions(emotions, wm, b)
    tok_w = tokens.T.reshape(_L, _NW, _BW).transpose(1, 0, 2)
    out = _sc_gather_add(tok_w, emo_proj, proj)
    return out.reshape(_L, _B, _D).transpose(1, 0, 2)


# speedup vs baseline: 1.2428x; 1.1202x over previous
import functools

import jax
import jax.numpy as jnp
from jax import lax
from jax.experimental import pallas as pl
from jax.experimental.pallas import tpu as pltpu
from jax.experimental.pallas import tpu_sc as plsc

_B = 4096
_L = 50
_V = 100000
_D = 128
_R = _B * _L

_NW = 32
_BW = _B // _NW
_NCHUNK = _L
_NBUF = 5
_UNROLL = 1


@functools.partial(
    pl.kernel,
    out_type=jax.ShapeDtypeStruct((_R, _D), jnp.float32),
    mesh=plsc.VectorSubcoreMesh(core_axis_name="c", subcore_axis_name="s"),
    scratch_types=[
        pltpu.VMEM((_NCHUNK, _BW), jnp.int32),
        pltpu.VMEM((1, _BW), jnp.int32),
        pltpu.VMEM_SHARED((_B, _D), jnp.float32),
        pltpu.VMEM((_NBUF, _BW, _D), jnp.float32),
        pltpu.SemaphoreType.DMA((_NBUF,)),
        pltpu.SemaphoreType.DMA((_NBUF,)),
        pltpu.SemaphoreType.DMA((_NBUF,)),
    ],
)
def _sc_gather_add(tok_hbm, eidx_hbm, emo_hbm, proj_hbm, out_hbm,
                   idx_v, eidx_v, emo_sh, rows_v, gsem, esem, ssem):
    sid = lax.axis_index("s")
    w = sid * 2 + lax.axis_index("c")
    pltpu.sync_copy(tok_hbm.at[w], idx_v)
    pltpu.sync_copy(eidx_hbm.at[w], eidx_v)

    @pl.when(sid == 0)
    def _stage_emo():
        pltpu.sync_copy(emo_hbm, emo_sh)

    plsc.subcore_barrier()

    def start_gather(j, s):
        pltpu.async_copy(proj_hbm.at[idx_v.at[j]], rows_v.at[s], gsem.at[s])

    for s in range(_NBUF - 1):
        start_gather(s, s)

    def ring_body(jj, _):
        for s in range(_NBUF):
            j = jj * _NBUF + s
            sn = (s + _NBUF - 1) % _NBUF

            @pl.when(j >= 1)
            def _wait_prev_store():
                pltpu.make_async_copy(
                    rows_v.at[sn], out_hbm.at[pl.ds(0, _BW)],
                    ssem.at[sn]).wait()

            @pl.when(j + _NBUF - 1 < _NCHUNK)
            def _next_gather():
                start_gather(j + _NBUF - 1, sn)

            pltpu.make_async_copy(
                proj_hbm.at[idx_v.at[j]], rows_v.at[s], gsem.at[s]).wait()
            pltpu.async_copy(
                emo_sh.at[eidx_v.at[0]], rows_v.at[s], esem.at[s],
                add=True)
            pltpu.make_async_copy(
                emo_sh.at[eidx_v.at[0]], rows_v.at[s], esem.at[s]).wait()
            pltpu.async_copy(
                rows_v.at[s],
                out_hbm.at[pl.ds(j * _B + w * _BW, _BW)],
                ssem.at[s])
        return _

    lax.fori_loop(0, _NCHUNK // _NBUF, ring_body, None)
    pltpu.make_async_copy(
        rows_v.at[_NBUF - 1], out_hbm.at[pl.ds(0, _BW)],
        ssem.at[_NBUF - 1]).wait()


def _tc_project_table(x, w):
    m = x.shape[0]
    blk = 4000

    def body(x_ref, w_ref, o_ref):
        o_ref[...] = lax.dot_general(
            x_ref[...], w_ref[...], (((1,), (1,)), ((), ())),
            preferred_element_type=jnp.float32)

    return pl.pallas_call(
        body,
        grid=(m // blk,),
        in_specs=[
            pl.BlockSpec((blk, _D), lambda i: (i, 0)),
            pl.BlockSpec((_D, _D), lambda i: (0, 0)),
        ],
        out_specs=pl.BlockSpec((blk, _D), lambda i: (i, 0)),
        out_shape=jax.ShapeDtypeStruct((m, _D), jnp.float32),
    )(x, w)


def _tc_project_emotions(x, w, bias):
    m = x.shape[0]

    def body(x_ref, w_ref, b_ref, o_ref):
        o_ref[...] = lax.dot_general(
            x_ref[...], w_ref[...], (((1,), (1,)), ((), ())),
            preferred_element_type=jnp.float32) + b_ref[...]

    return pl.pallas_call(
        body,
        grid=(1,),
        in_specs=[
            pl.BlockSpec((m, _D), lambda i: (0, 0)),
            pl.BlockSpec((_D, _D), lambda i: (0, 0)),
            pl.BlockSpec((1, _D), lambda i: (0, 0)),
        ],
        out_specs=pl.BlockSpec((m, _D), lambda i: (0, 0)),
        out_shape=jax.ShapeDtypeStruct((m, _D), jnp.float32),
    )(x, w, bias.reshape(1, _D))


def kernel(tokens, emotions, table, W, b):
    tokens = tokens.astype(jnp.int32)
    we = W[:, :_D]
    wm = W[:, _D:]

    proj = _tc_project_table(table, we)
    emo_proj = _tc_project_emotions(emotions, wm, b)
    tok_w = tokens.T.reshape(_L, _NW, _BW).transpose(1, 0, 2)
    eidx = jnp.arange(_B, dtype=jnp.int32).reshape(_NW, 1, _BW)
    out = _sc_gather_add(tok_w, eidx, emo_proj, proj)
    return out.reshape(_L, _B, _D).transpose(1, 0, 2)


# speedup vs baseline: 1.2972x; 1.0438x over previous
import functools

import jax
import jax.numpy as jnp
from jax import lax
from jax.experimental import pallas as pl
from jax.experimental.pallas import tpu as pltpu
from jax.experimental.pallas import tpu_sc as plsc

_B = 4096
_L = 50
_V = 100000
_D = 128
_R = _B * _L

_NW = 32
_BW = _B // _NW
_NCHUNK = _L
_NBUF = 5
_UNROLL = 1


@functools.partial(
    pl.kernel,
    out_type=jax.ShapeDtypeStruct((_R, _D), jnp.float32),
    mesh=plsc.VectorSubcoreMesh(core_axis_name="c", subcore_axis_name="s"),
    scratch_types=[
        pltpu.VMEM((_NCHUNK, _BW), jnp.int32),
        pltpu.VMEM((1, _BW), jnp.int32),
        pltpu.VMEM_SHARED((_B, _D), jnp.float32),
        pltpu.VMEM((_NBUF, _BW, _D), jnp.float32),
        pltpu.SemaphoreType.DMA((_NBUF,)),
        pltpu.SemaphoreType.DMA((_NBUF,)),
        pltpu.SemaphoreType.DMA((_NBUF,)),
    ],
)
def _sc_gather_add(tok_hbm, eidx_hbm, emo_hbm, proj_hbm, out_hbm,
                   idx_v, eidx_v, emo_sh, rows_v, gsem, esem, ssem):
    sid = lax.axis_index("s")
    w = sid * 2 + lax.axis_index("c")
    pltpu.sync_copy(tok_hbm.at[w], idx_v)
    pltpu.sync_copy(eidx_hbm.at[w], eidx_v)

    @pl.when(sid == 0)
    def _stage_emo():
        pltpu.sync_copy(emo_hbm, emo_sh)

    plsc.subcore_barrier()

    def start_gather(j, s):
        pltpu.async_copy(proj_hbm.at[idx_v.at[j]], rows_v.at[s], gsem.at[s])

    for s in range(_NBUF - 1):
        start_gather(s, s)

    def ring_body(jj, _):
        for s in range(_NBUF):
            j = jj * _NBUF + s
            sn = (s + _NBUF - 1) % _NBUF

            @pl.when(j >= 1)
            def _wait_prev_store():
                pltpu.make_async_copy(
                    rows_v.at[sn], out_hbm.at[pl.ds(0, _BW)],
                    ssem.at[sn]).wait()

            @pl.when(j + _NBUF - 1 < _NCHUNK)
            def _next_gather():
                start_gather(j + _NBUF - 1, sn)

            pltpu.make_async_copy(
                proj_hbm.at[idx_v.at[j]], rows_v.at[s], gsem.at[s]).wait()
            pltpu.async_copy(
                emo_sh.at[eidx_v.at[0]], rows_v.at[s], esem.at[s],
                add=True)
            pltpu.make_async_copy(
                emo_sh.at[eidx_v.at[0]], rows_v.at[s], esem.at[s]).wait()
            pltpu.async_copy(
                rows_v.at[s],
                out_hbm.at[pl.ds(j * _B + w * _BW, _BW)],
                ssem.at[s])
        return _

    lax.fori_loop(0, _NCHUNK // _NBUF, ring_body, None)
    pltpu.make_async_copy(
        rows_v.at[_NBUF - 1], out_hbm.at[pl.ds(0, _BW)],
        ssem.at[_NBUF - 1]).wait()


def _tc_project_table(x, w):
    m = x.shape[0]
    blk = 10000

    def body(x_ref, w_ref, o_ref):
        o_ref[...] = lax.dot_general(
            x_ref[...], w_ref[...], (((1,), (1,)), ((), ())),
            preferred_element_type=jnp.float32)

    return pl.pallas_call(
        body,
        grid=(m // blk,),
        in_specs=[
            pl.BlockSpec((blk, _D), lambda i: (i, 0)),
            pl.BlockSpec((_D, _D), lambda i: (0, 0)),
        ],
        out_specs=pl.BlockSpec((blk, _D), lambda i: (i, 0)),
        out_shape=jax.ShapeDtypeStruct((m, _D), jnp.float32),
    )(x, w)


def _tc_project_emotions(x, w, bias):
    m = x.shape[0]

    def body(x_ref, w_ref, b_ref, o_ref):
        o_ref[...] = lax.dot_general(
            x_ref[...], w_ref[...], (((1,), (1,)), ((), ())),
            preferred_element_type=jnp.float32) + b_ref[...]

    return pl.pallas_call(
        body,
        grid=(1,),
        in_specs=[
            pl.BlockSpec((m, _D), lambda i: (0, 0)),
            pl.BlockSpec((_D, _D), lambda i: (0, 0)),
            pl.BlockSpec((1, _D), lambda i: (0, 0)),
        ],
        out_specs=pl.BlockSpec((m, _D), lambda i: (0, 0)),
        out_shape=jax.ShapeDtypeStruct((m, _D), jnp.float32),
    )(x, w, bias.reshape(1, _D))


def kernel(tokens, emotions, table, W, b):
    tokens = tokens.astype(jnp.int32)
    we = W[:, :_D]
    wm = W[:, _D:]

    proj = _tc_project_table(table, we)
    emo_proj = _tc_project_emotions(emotions, wm, b)
    tok_w = tokens.T.reshape(_L, _NW, _BW).transpose(1, 0, 2)
    eidx = jnp.arange(_B, dtype=jnp.int32).reshape(_NW, 1, _BW)
    out = _sc_gather_add(tok_w, eidx, emo_proj, proj)
    return out.reshape(_L, _B, _D).transpose(1, 0, 2)


# speedup vs baseline: 1.3114x; 1.0109x over previous
import functools

import jax
import jax.numpy as jnp
from jax import lax
from jax.experimental import pallas as pl
from jax.experimental.pallas import tpu as pltpu
from jax.experimental.pallas import tpu_sc as plsc

_B = 4096
_L = 50
_V = 100000
_D = 128
_R = _B * _L

_NW = 32
_BW = _B // _NW
_NCHUNK = _L
_NBUF = 5
_UNROLL = 1


@functools.partial(
    pl.kernel,
    out_type=jax.ShapeDtypeStruct((_R, _D), jnp.float32),
    mesh=plsc.VectorSubcoreMesh(core_axis_name="c", subcore_axis_name="s"),
    scratch_types=[
        pltpu.VMEM((_NCHUNK, _BW), jnp.int32),
        pltpu.VMEM((1, _BW), jnp.int32),
        pltpu.VMEM_SHARED((_B, _D), jnp.float32),
        pltpu.VMEM((_NBUF, _BW, _D), jnp.float32),
        pltpu.SemaphoreType.DMA((_NBUF,)),
        pltpu.SemaphoreType.DMA((_NBUF,)),
        pltpu.SemaphoreType.DMA((_NBUF,)),
    ],
)
def _sc_gather_add(tok_hbm, eidx_hbm, emo_hbm, proj_hbm, out_hbm,
                   idx_v, eidx_v, emo_sh, rows_v, gsem, esem, ssem):
    sid = lax.axis_index("s")
    w = sid * 2 + lax.axis_index("c")
    pltpu.sync_copy(tok_hbm.at[w], idx_v)
    pltpu.sync_copy(eidx_hbm.at[w], eidx_v)

    @pl.when(sid == 0)
    def _stage_emo():
        pltpu.sync_copy(emo_hbm, emo_sh)

    plsc.subcore_barrier()

    def start_gather(j, s):
        pltpu.async_copy(proj_hbm.at[idx_v.at[j]], rows_v.at[s], gsem.at[s])

    for s in range(_NBUF - 1):
        start_gather(s, s)

    def ring_body(jj, _):
        for s in range(_NBUF):
            j = jj * _NBUF + s
            sn = (s + _NBUF - 1) % _NBUF

            @pl.when(j >= 1)
            def _wait_prev_store():
                pltpu.make_async_copy(
                    rows_v.at[sn], out_hbm.at[pl.ds(0, _BW)],
                    ssem.at[sn]).wait()

            @pl.when(j + _NBUF - 1 < _NCHUNK)
            def _next_gather():
                start_gather(j + _NBUF - 1, sn)

            pltpu.make_async_copy(
                proj_hbm.at[idx_v.at[j]], rows_v.at[s], gsem.at[s]).wait()
            pltpu.async_copy(
                emo_sh.at[eidx_v.at[0]], rows_v.at[s], esem.at[s],
                add=True)
            pltpu.make_async_copy(
                emo_sh.at[eidx_v.at[0]], rows_v.at[s], esem.at[s]).wait()
            pltpu.async_copy(
                rows_v.at[s],
                out_hbm.at[pl.ds(j * _B + w * _BW, _BW)],
                ssem.at[s])
        return _

    lax.fori_loop(0, _NCHUNK // _NBUF, ring_body, None)
    pltpu.make_async_copy(
        rows_v.at[_NBUF - 1], out_hbm.at[pl.ds(0, _BW)],
        ssem.at[_NBUF - 1]).wait()


def _tc_project_table(x, w):
    m = x.shape[0]
    blk = 20000

    def body(x_ref, w_ref, o_ref):
        o_ref[...] = lax.dot_general(
            x_ref[...], w_ref[...], (((1,), (1,)), ((), ())),
            preferred_element_type=jnp.float32)

    return pl.pallas_call(
        body,
        grid=(m // blk,),
        in_specs=[
            pl.BlockSpec((blk, _D), lambda i: (i, 0)),
            pl.BlockSpec((_D, _D), lambda i: (0, 0)),
        ],
        out_specs=pl.BlockSpec((blk, _D), lambda i: (i, 0)),
        out_shape=jax.ShapeDtypeStruct((m, _D), jnp.float32),
    )(x, w)


def _tc_project_emotions(x, w, bias):
    m = x.shape[0]

    def body(x_ref, w_ref, b_ref, o_ref):
        o_ref[...] = lax.dot_general(
            x_ref[...], w_ref[...], (((1,), (1,)), ((), ())),
            preferred_element_type=jnp.float32) + b_ref[...]

    return pl.pallas_call(
        body,
        grid=(1,),
        in_specs=[
            pl.BlockSpec((m, _D), lambda i: (0, 0)),
            pl.BlockSpec((_D, _D), lambda i: (0, 0)),
            pl.BlockSpec((1, _D), lambda i: (0, 0)),
        ],
        out_specs=pl.BlockSpec((m, _D), lambda i: (0, 0)),
        out_shape=jax.ShapeDtypeStruct((m, _D), jnp.float32),
    )(x, w, bias.reshape(1, _D))


def kernel(tokens, emotions, table, W, b):
    tokens = tokens.astype(jnp.int32)
    we = W[:, :_D]
    wm = W[:, _D:]

    proj = _tc_project_table(table, we)
    emo_proj = _tc_project_emotions(emotions, wm, b)
    tok_w = tokens.T.reshape(_L, _NW, _BW).transpose(1, 0, 2)
    eidx = jnp.arange(_B, dtype=jnp.int32).reshape(_NW, 1, _BW)
    out = _sc_gather_add(tok_w, eidx, emo_proj, proj)
    return out.reshape(_L, _B, _D).transpose(1, 0, 2)


# speedup vs baseline: 1.3255x; 1.0108x over previous
import functools

import jax
import jax.numpy as jnp
from jax import lax
from jax.experimental import pallas as pl
from jax.experimental.pallas import tpu as pltpu
from jax.experimental.pallas import tpu_sc as plsc

_B = 4096
_L = 50
_V = 100000
_D = 128
_R = _B * _L

_NW = 32
_BW = _B // _NW
_NCHUNK = _L
_NBUF = 5
_UNROLL = 1


@functools.partial(
    pl.kernel,
    out_type=jax.ShapeDtypeStruct((_R, _D), jnp.float32),
    mesh=plsc.VectorSubcoreMesh(core_axis_name="c", subcore_axis_name="s"),
    scratch_types=[
        pltpu.VMEM((_NCHUNK, _BW), jnp.int32),
        pltpu.VMEM((1, _BW), jnp.int32),
        pltpu.VMEM_SHARED((_B, _D), jnp.float32),
        pltpu.VMEM((_NBUF, _BW, _D), jnp.float32),
        pltpu.SemaphoreType.DMA((_NBUF,)),
        pltpu.SemaphoreType.DMA((_NBUF,)),
        pltpu.SemaphoreType.DMA((_NBUF,)),
    ],
)
def _sc_gather_add(tok_hbm, eidx_hbm, emo_hbm, proj_hbm, out_hbm,
                   idx_v, eidx_v, emo_sh, rows_v, gsem, esem, ssem):
    sid = lax.axis_index("s")
    w = sid * 2 + lax.axis_index("c")
    pltpu.sync_copy(tok_hbm.at[w], idx_v)
    pltpu.sync_copy(eidx_hbm.at[w], eidx_v)

    @pl.when(sid == 0)
    def _stage_emo():
        pltpu.sync_copy(emo_hbm, emo_sh)

    plsc.subcore_barrier()

    def start_gather(j, s):
        pltpu.async_copy(proj_hbm.at[idx_v.at[j]], rows_v.at[s], gsem.at[s])

    for s in range(_NBUF - 1):
        start_gather(s, s)

    def ring_body(jj, _):
        for s in range(_NBUF):
            j = jj * _NBUF + s
            sn = (s + _NBUF - 1) % _NBUF

            @pl.when(j >= 1)
            def _wait_prev_store():
                pltpu.make_async_copy(
                    rows_v.at[sn], out_hbm.at[pl.ds(0, _BW)],
                    ssem.at[sn]).wait()

            @pl.when(j + _NBUF - 1 < _NCHUNK)
            def _next_gather():
                start_gather(j + _NBUF - 1, sn)

            pltpu.make_async_copy(
                proj_hbm.at[idx_v.at[j]], rows_v.at[s], gsem.at[s]).wait()
            pltpu.async_copy(
                emo_sh.at[eidx_v.at[0]], rows_v.at[s], esem.at[s],
                add=True)
            pltpu.make_async_copy(
                emo_sh.at[eidx_v.at[0]], rows_v.at[s], esem.at[s]).wait()
            pltpu.async_copy(
                rows_v.at[s],
                out_hbm.at[pl.ds(j * _B + w * _BW, _BW)],
                ssem.at[s])
        return _

    lax.fori_loop(0, _NCHUNK // _NBUF, ring_body, None)
    pltpu.make_async_copy(
        rows_v.at[_NBUF - 1], out_hbm.at[pl.ds(0, _BW)],
        ssem.at[_NBUF - 1]).wait()


def _tc_project(table, we, emotions, wm, bias):
    m = table.shape[0]
    blk = 20000

    def body(x_ref, w_ref, e_ref, wm_ref, b_ref, o_ref, eo_ref):
        o_ref[...] = lax.dot_general(
            x_ref[...], w_ref[...], (((1,), (1,)), ((), ())),
            preferred_element_type=jnp.float32)

        @pl.when(pl.program_id(0) == 0)
        def _emo():
            eo_ref[...] = lax.dot_general(
                e_ref[...], wm_ref[...], (((1,), (1,)), ((), ())),
                preferred_element_type=jnp.float32) + b_ref[...]

    return pl.pallas_call(
        body,
        grid=(m // blk,),
        in_specs=[
            pl.BlockSpec((blk, _D), lambda i: (i, 0)),
            pl.BlockSpec((_D, _D), lambda i: (0, 0)),
            pl.BlockSpec((_B, _D), lambda i: (0, 0)),
            pl.BlockSpec((_D, _D), lambda i: (0, 0)),
            pl.BlockSpec((1, _D), lambda i: (0, 0)),
        ],
        out_specs=[
            pl.BlockSpec((blk, _D), lambda i: (i, 0)),
            pl.BlockSpec((_B, _D), lambda i: (0, 0)),
        ],
        out_shape=[
            jax.ShapeDtypeStruct((m, _D), jnp.float32),
            jax.ShapeDtypeStruct((_B, _D), jnp.float32),
        ],
    )(table, we, emotions, wm, bias.reshape(1, _D))


def kernel(tokens, emotions, table, W, b):
    tokens = tokens.astype(jnp.int32)
    we = W[:, :_D]
    wm = W[:, _D:]

    proj, emo_proj = _tc_project(table, we, emotions, wm, b)
    tok_w = tokens.T.reshape(_L, _NW, _BW).transpose(1, 0, 2)
    eidx = jnp.arange(_B, dtype=jnp.int32).reshape(_NW, 1, _BW)
    out = _sc_gather_add(tok_w, eidx, emo_proj, proj)
    return out.reshape(_L, _B, _D).transpose(1, 0, 2)
